# Initial kernel scaffold; baseline (speedup 1.0000x reference)
import jax
import jax.numpy as jnp
from jax.experimental import pallas as pl


def kernel(sim_matrix):
    raise NotImplementedError("write your pallas kernel here")


# speedup vs baseline: 7.0691x; 7.0691x over previous
import jax
import jax.numpy as jnp
from jax.experimental import pallas as pl
from jax.experimental.pallas import tpu as pltpu

_NQ = 32768
_NT = 128
_TOPK = 10
_POS = 5
_CHUNK = 256


def _topk_kernel(sim_ref, vals_ref, ids_ref, mprev_ref, qprev_ref):
    t = pl.program_id(0)

    @pl.when(t == 0)
    def _():
        mprev_ref[...] = jnp.full((1, _NT), jnp.inf, jnp.float32)
        qprev_ref[...] = jnp.full((1, _NT), -1, jnp.int32)

    mp = mprev_ref[...]
    qp = qprev_ref[...]

    def body(i, carry):
        m, q = carry
        base = i * _CHUNK
        blk = sim_ref[pl.ds(base, _CHUNK), :]
        riota = jax.lax.broadcasted_iota(jnp.int32, (_CHUNK, _NT), 0) + base
        elig = (blk < mp) | ((blk == mp) & (riota > qp))
        mb = jnp.where(elig, blk, -jnp.inf)
        bm = jnp.max(mb, axis=0, keepdims=True)
        bq = jnp.min(jnp.where(mb == bm, riota, 2**30), axis=0, keepdims=True)
        take = (bm > m) | ((bm == m) & (bq < q))
        return jnp.where(take, bm, m), jnp.where(take, bq, q)

    m0 = jnp.full((1, _NT), -jnp.inf, jnp.float32)
    q0 = jnp.full((1, _NT), 2**30, jnp.int32)
    m, q = jax.lax.fori_loop(0, _NQ // _CHUNK, body, (m0, q0))
    vals_ref[pl.ds(t, 1), :] = m
    ids_ref[pl.ds(t, 1), :] = q
    mprev_ref[...] = m
    qprev_ref[...] = q


def _labels_kernel(sim_ref, vals_ref, ids_ref, out_ref):
    b = pl.program_id(0)
    rows = sim_ref.shape[0]
    blk = sim_ref[...]
    riota = jax.lax.broadcasted_iota(jnp.int32, (rows, _NT), 0) + b * rows
    v5 = vals_ref[_POS - 1:_POS, :]
    q5 = ids_ref[_POS - 1:_POS, :]
    v10 = vals_ref[_TOPK - 1:_TOPK, :]
    q10 = ids_ref[_TOPK - 1:_TOPK, :]
    in5 = (blk > v5) | ((blk == v5) & (riota <= q5))
    in10 = (blk > v10) | ((blk == v10) & (riota <= q10))
    any5 = jnp.max(jnp.where(in5, 1, 0), axis=1, keepdims=True)
    any10 = jnp.max(jnp.where(in10, 1, 0), axis=1, keepdims=True)
    out_ref[...] = jnp.where(any5 > 0, 1, jnp.where(any10 > 0, -1, 0)).astype(jnp.int32)


def _pairs_kernel(ids_ref, qry_ref, tgt_ref):
    ids8 = ids_ref[0:8, :]
    row = jax.lax.broadcasted_iota(jnp.int32, (8, _NT), 0)
    col = jax.lax.broadcasted_iota(jnp.int32, (8, _NT), 1)
    keyf = ids8.astype(jnp.float32) * jnp.float32(_NT) + col.astype(jnp.float32)
    K = jnp.where(row < _POS, keyf, jnp.float32(1e9))

    eye = (jax.lax.broadcasted_iota(jnp.int32, (_NT, _NT), 0)
           == jax.lax.broadcasted_iota(jnp.int32, (_NT, _NT), 1)).astype(jnp.float32)
    KT = jax.lax.dot_general(eye, K, (((1,), (1,)), ((), ())),
                             precision=jax.lax.Precision.HIGHEST,
                             preferred_element_type=jnp.float32)

    lane = jax.lax.broadcasted_iota(jnp.int32, (1, _NT), 1).astype(jnp.float32)
    rank_cols = []
    for r in range(_POS):
        kcol = KT[:, r:r + 1]
        cnt = jnp.zeros((_NT, 1), jnp.float32)
        for rp in range(_POS):
            cnt = cnt + jnp.sum((K[rp:rp + 1, :] < kcol).astype(jnp.float32),
                                axis=1, keepdims=True)
        rank_cols.append(cnt)

    qrows, trows = [], []
    for i in range(_POS):
        accq = jnp.zeros((1, _NT), jnp.float32)
        acct = jnp.zeros((1, _NT), jnp.float32)
        for r in range(_POS):
            onehot = (rank_cols[r] - jnp.float32(_NT * i) == lane).astype(jnp.float32)
            vq = ids8[r:r + 1, :].astype(jnp.float32)
            accq = accq + jax.lax.dot_general(
                vq, onehot, (((1,), (0,)), ((), ())),
                precision=jax.lax.Precision.HIGHEST, preferred_element_type=jnp.float32)
            acct = acct + jax.lax.dot_general(
                lane, onehot, (((1,), (0,)), ((), ())),
                precision=jax.lax.Precision.HIGHEST, preferred_element_type=jnp.float32)
        qrows.append(accq)
        trows.append(acct)
    pad = jnp.zeros((8 - _POS, _NT), jnp.float32)
    qry_ref[...] = jnp.concatenate(qrows + [pad], axis=0).astype(jnp.int32)
    tgt_ref[...] = jnp.concatenate(trows + [pad], axis=0).astype(jnp.int32)


def kernel(sim_matrix):
    vals, ids = pl.pallas_call(
        _topk_kernel,
        grid=(_TOPK,),
        in_specs=[pl.BlockSpec((_NQ, _NT), lambda t: (0, 0))],
        out_specs=[pl.BlockSpec((16, _NT), lambda t: (0, 0)),
                   pl.BlockSpec((16, _NT), lambda t: (0, 0))],
        out_shape=[jax.ShapeDtypeStruct((16, _NT), jnp.float32),
                   jax.ShapeDtypeStruct((16, _NT), jnp.int32)],
        scratch_shapes=[pltpu.VMEM((1, _NT), jnp.float32),
                        pltpu.VMEM((1, _NT), jnp.int32)],
    )(sim_matrix)

    rows_per_blk = 1024
    labels2 = pl.pallas_call(
        _labels_kernel,
        grid=(_NQ // rows_per_blk,),
        in_specs=[pl.BlockSpec((rows_per_blk, _NT), lambda b: (b, 0)),
                  pl.BlockSpec((16, _NT), lambda b: (0, 0)),
                  pl.BlockSpec((16, _NT), lambda b: (0, 0))],
        out_specs=pl.BlockSpec((rows_per_blk, 1), lambda b: (b, 0)),
        out_shape=jax.ShapeDtypeStruct((_NQ, 1), jnp.int32),
    )(sim_matrix, vals, ids)

    qry2, tgt2 = pl.pallas_call(
        _pairs_kernel,
        out_shape=[jax.ShapeDtypeStruct((8, _NT), jnp.int32),
                   jax.ShapeDtypeStruct((8, _NT), jnp.int32)],
    )(ids)

    labels = labels2[:, 0]
    qry = qry2.reshape(8 * _NT)[:_POS * _NT]
    tgt = tgt2.reshape(8 * _NT)[:_POS * _NT]
    return labels, qry, tgt


# speedup vs baseline: 7.9483x; 1.1244x over previous
import functools

import jax
import jax.numpy as jnp
from jax import lax
from jax.experimental import pallas as pl
from jax.experimental.pallas import tpu as pltpu
from jax.experimental.pallas import tpu_sc as plsc

_NQ = 32768
_NT = 128
_TOPK = 10
_POS = 5
_CHUNK = 256


def _topk_kernel(sim_ref, vals_ref, ids_ref, mprev_ref, qprev_ref):
    t = pl.program_id(0)

    @pl.when(t == 0)
    def _():
        mprev_ref[...] = jnp.full((1, _NT), jnp.inf, jnp.float32)
        qprev_ref[...] = jnp.full((1, _NT), -1, jnp.int32)

    mp = mprev_ref[...]
    qp = qprev_ref[...]

    def body(i, carry):
        m, q = carry
        base = i * _CHUNK
        blk = sim_ref[pl.ds(base, _CHUNK), :]
        riota = jax.lax.broadcasted_iota(jnp.int32, (_CHUNK, _NT), 0) + base
        elig = (blk < mp) | ((blk == mp) & (riota > qp))
        mb = jnp.where(elig, blk, -jnp.inf)
        bm = jnp.max(mb, axis=0, keepdims=True)
        bq = jnp.min(jnp.where(mb == bm, riota, 2**30), axis=0, keepdims=True)
        take = (bm > m) | ((bm == m) & (bq < q))
        return jnp.where(take, bm, m), jnp.where(take, bq, q)

    m0 = jnp.full((1, _NT), -jnp.inf, jnp.float32)
    q0 = jnp.full((1, _NT), 2**30, jnp.int32)
    m, q = jax.lax.fori_loop(0, _NQ // _CHUNK, body, (m0, q0))
    vals_ref[pl.ds(t, 1), :] = m
    ids_ref[pl.ds(t, 1), :] = q
    mprev_ref[...] = m
    qprev_ref[...] = q


def _labels_kernel(sim_ref, vals_ref, ids_ref, out_ref):
    b = pl.program_id(0)
    rows = sim_ref.shape[0]
    blk = sim_ref[...]
    riota = jax.lax.broadcasted_iota(jnp.int32, (rows, _NT), 0) + b * rows
    v5 = vals_ref[_POS - 1:_POS, :]
    q5 = ids_ref[_POS - 1:_POS, :]
    v10 = vals_ref[_TOPK - 1:_TOPK, :]
    q10 = ids_ref[_TOPK - 1:_TOPK, :]
    in5 = (blk > v5) | ((blk == v5) & (riota <= q5))
    in10 = (blk > v10) | ((blk == v10) & (riota <= q10))
    any5 = jnp.max(jnp.where(in5, 1, 0), axis=1, keepdims=True)
    any10 = jnp.max(jnp.where(in10, 1, 0), axis=1, keepdims=True)
    out_ref[...] = jnp.where(any5 > 0, 1, jnp.where(any10 > 0, -1, 0)).astype(jnp.int32)


def _pairs_kernel(ids_ref, qry_ref, tgt_ref):
    ids8 = ids_ref[0:8, :]
    row = jax.lax.broadcasted_iota(jnp.int32, (8, _NT), 0)
    col = jax.lax.broadcasted_iota(jnp.int32, (8, _NT), 1)
    keyf = ids8.astype(jnp.float32) * jnp.float32(_NT) + col.astype(jnp.float32)
    K = jnp.where(row < _POS, keyf, jnp.float32(1e9))

    eye = (jax.lax.broadcasted_iota(jnp.int32, (_NT, _NT), 0)
           == jax.lax.broadcasted_iota(jnp.int32, (_NT, _NT), 1)).astype(jnp.float32)
    KT = jax.lax.dot_general(eye, K, (((1,), (1,)), ((), ())),
                             precision=jax.lax.Precision.HIGHEST,
                             preferred_element_type=jnp.float32)

    lane = jax.lax.broadcasted_iota(jnp.int32, (1, _NT), 1).astype(jnp.float32)
    rank_cols = []
    for r in range(_POS):
        kcol = KT[:, r:r + 1]
        cnt = jnp.zeros((_NT, 1), jnp.float32)
        for rp in range(_POS):
            cnt = cnt + jnp.sum((K[rp:rp + 1, :] < kcol).astype(jnp.float32),
                                axis=1, keepdims=True)
        rank_cols.append(cnt)

    qrows, trows = [], []
    for i in range(_POS):
        accq = jnp.zeros((1, _NT), jnp.float32)
        acct = jnp.zeros((1, _NT), jnp.float32)
        for r in range(_POS):
            onehot = (rank_cols[r] - jnp.float32(_NT * i) == lane).astype(jnp.float32)
            vq = ids8[r:r + 1, :].astype(jnp.float32)
            accq = accq + jax.lax.dot_general(
                vq, onehot, (((1,), (0,)), ((), ())),
                precision=jax.lax.Precision.HIGHEST, preferred_element_type=jnp.float32)
            acct = acct + jax.lax.dot_general(
                lane, onehot, (((1,), (0,)), ((), ())),
                precision=jax.lax.Precision.HIGHEST, preferred_element_type=jnp.float32)
        qrows.append(accq)
        trows.append(acct)
    pad = jnp.zeros((8 - _POS, _NT), jnp.float32)
    qry_ref[...] = jnp.concatenate(qrows + [pad], axis=0).astype(jnp.int32)
    tgt_ref[...] = jnp.concatenate(trows + [pad], axis=0).astype(jnp.int32)


_N_PAIR = _POS * _NT
_QSLICE = _NQ // 16


def _sc_mesh():
    return plsc.VectorSubcoreMesh(core_axis_name="c", subcore_axis_name="s")


def _sc_post_body(idsT_hbm, labels_hbm, qry_hbm, tgt_hbm,
                  idsT_v, lab_v, keys_v, stage_v, rankall_v, qry_v, tgt_v,
                  rank_sh):
    c = lax.axis_index("c")
    s = lax.axis_index("s")
    iota = lax.iota(jnp.int32, 16)

    @pl.when(c == 0)
    def _():
        pltpu.sync_copy(idsT_hbm, idsT_v)
        lo = s * _QSLICE

        zeros16 = jnp.zeros((16,), jnp.int32)

        def zero_body(i, _):
            plsc.store_scatter(lab_v, [iota + i * 16], zeros16)
            return 0
        lax.fori_loop(0, _QSLICE // 16, zero_body, 0)

        def scatter_phase(col, carry):
            neg_phase = carry
            ids16 = plsc.load_gather(idsT_v, [jnp.full((16,), col, jnp.int32), iota])
            ids16 = ids16 & (_NQ - 1)
            inrange = (ids16 >= lo) & (ids16 < lo + _QSLICE)
            lane_ok = jnp.where(neg_phase == 1, (iota >= _POS) & (iota < _TOPK),
                                iota < _POS)
            val = jnp.full((16,), 0, jnp.int32) + jnp.where(neg_phase == 1, -1, 1)
            plsc.store_scatter(lab_v, [ids16 - lo], val, mask=inrange & lane_ok)
            return carry

        lax.fori_loop(0, _NT, scatter_phase, jnp.int32(1))
        lax.fori_loop(0, _NT, scatter_phase, jnp.int32(0))
        pltpu.sync_copy(lab_v, labels_hbm.at[pl.ds(lo, _QSLICE)])

        def key_body(k, _):
            j = iota + k * 16
            colv = j // _POS
            rankv = j - colv * _POS
            g = plsc.load_gather(idsT_v, [colv, rankv])
            plsc.store_scatter(keys_v, [j], g * _NT + colv)
            return 0
        lax.fori_loop(0, _N_PAIR // 16, key_body, 0)

        def rank_one(k):
            a = plsc.load_gather(keys_v, [iota + k * 16])

            def inner(t, acc):
                b = t // 16
                r = t - b * 16
                rot = plsc.load_gather(keys_v, [b * 16 + ((iota + r) & 15)])
                return acc + jnp.where(rot < a, 1, 0).astype(jnp.int32)

            rank = lax.fori_loop(0, _N_PAIR, inner, jnp.zeros((16,), jnp.int32))
            stage_v[...] = rank
            pltpu.sync_copy(stage_v, rank_sh.at[pl.ds(k * 16, 16)])

        n_vec = _N_PAIR // 16
        for m in range((n_vec + 15) // 16):
            k = s + m * 16

            @pl.when(k < n_vec)
            def _():
                rank_one(k)

        plsc.subcore_barrier()

        @pl.when(s == 0)
        def _():
            pltpu.sync_copy(rank_sh, rankall_v)

            def place(k, _):
                keys = plsc.load_gather(keys_v, [iota + k * 16])
                rk = plsc.load_gather(rankall_v, [iota + k * 16])
                plsc.store_scatter(qry_v, [rk], keys >> 7)
                plsc.store_scatter(tgt_v, [rk], keys & (_NT - 1))
                return 0
            lax.fori_loop(0, n_vec, place, 0)
            pltpu.sync_copy(qry_v, qry_hbm)
            pltpu.sync_copy(tgt_v, tgt_hbm)


def _sc_post(idsT):
    fn = functools.partial(
        pl.kernel,
        mesh=_sc_mesh(),
        out_type=[jax.ShapeDtypeStruct((_NQ,), jnp.int32),
                  jax.ShapeDtypeStruct((_N_PAIR,), jnp.int32),
                  jax.ShapeDtypeStruct((_N_PAIR,), jnp.int32)],
        scratch_types=[pltpu.VMEM((_NT, 16), jnp.int32),
                       pltpu.VMEM((_QSLICE,), jnp.int32),
                       pltpu.VMEM((_N_PAIR,), jnp.int32),
                       pltpu.VMEM((16,), jnp.int32),
                       pltpu.VMEM((_N_PAIR,), jnp.int32),
                       pltpu.VMEM((_N_PAIR,), jnp.int32),
                       pltpu.VMEM((_N_PAIR,), jnp.int32),
                       pltpu.VMEM_SHARED((_N_PAIR,), jnp.int32)],
        compiler_params=pltpu.CompilerParams(needs_layout_passes=False),
    )(_sc_post_body)
    return fn(idsT)


def kernel(sim_matrix):
    vals, ids = pl.pallas_call(
        _topk_kernel,
        grid=(_TOPK,),
        in_specs=[pl.BlockSpec((_NQ, _NT), lambda t: (0, 0))],
        out_specs=[pl.BlockSpec((16, _NT), lambda t: (0, 0)),
                   pl.BlockSpec((16, _NT), lambda t: (0, 0))],
        out_shape=[jax.ShapeDtypeStruct((16, _NT), jnp.float32),
                   jax.ShapeDtypeStruct((16, _NT), jnp.int32)],
        scratch_shapes=[pltpu.VMEM((1, _NT), jnp.float32),
                        pltpu.VMEM((1, _NT), jnp.int32)],
    )(sim_matrix)

    idsT = ids.T
    labels, qry, tgt = _sc_post(idsT)
    return labels, qry, tgt


# speedup vs baseline: 13.3644x; 1.6814x over previous
import functools

import jax
import jax.numpy as jnp
from jax import lax
from jax.experimental import pallas as pl
from jax.experimental.pallas import tpu as pltpu
from jax.experimental.pallas import tpu_sc as plsc

_NQ = 32768
_NT = 128
_TOPK = 10
_POS = 5
_NG = 4096
_NSLAB = _NQ // _NG
_MCHUNK = 256

_N_PAIR = _POS * _NT
_QSLICE = _NQ // 16
_NCAND = _TOPK * _NSLAB


def _gsel_kernel(sim_ref, sel_ref, m_ref, m2_ref, mprev_ref, qprev_ref):
    t = pl.program_id(0)

    @pl.when(t == 0)
    def _():
        mprev_ref[...] = jnp.full((1, _NT), jnp.inf, jnp.float32)
        qprev_ref[...] = jnp.full((1, _NT), -1, jnp.int32)

        def mbody(i, _):
            base = i * _MCHUNK
            giota = jax.lax.broadcasted_iota(jnp.int32, (_MCHUNK, _NT), 0) + base
            cur = sim_ref[pl.ds(base, _MCHUNK), :]
            curi = giota
            for j in range(1, _NSLAB):
                blk = sim_ref[pl.ds(base + j * _NG, _MCHUNK), :]
                take = blk > cur
                curi = jnp.where(take, giota + j * _NG, curi)
                cur = jnp.where(take, blk, cur)
            m_ref[pl.ds(base, _MCHUNK), :] = cur
            m2_ref[pl.ds(base, _MCHUNK), :] = curi
            return 0

        lax.fori_loop(0, _NG // _MCHUNK, mbody, 0)

    mp = mprev_ref[...]
    qp = qprev_ref[...]

    def body(i, carry):
        m, q = carry
        blk = m_ref[pl.ds(i * _MCHUNK, _MCHUNK), :]
        bli = m2_ref[pl.ds(i * _MCHUNK, _MCHUNK), :]
        elig = (blk < mp) | ((blk == mp) & (bli > qp))
        mb = jnp.where(elig, blk, -jnp.inf)
        bm = jnp.max(mb, axis=0, keepdims=True)
        bq = jnp.min(jnp.where(mb == bm, bli, 2**30), axis=0, keepdims=True)
        take = (bm > m) | ((bm == m) & (bq < q))
        return jnp.where(take, bm, m), jnp.where(take, bq, q)

    m0 = jnp.full((1, _NT), -jnp.inf, jnp.float32)
    q0 = jnp.full((1, _NT), 2**30, jnp.int32)
    m, q = lax.fori_loop(0, _NG // _MCHUNK, body, (m0, q0))
    sel_ref[pl.ds(t, 1), :] = q
    mprev_ref[...] = m
    qprev_ref[...] = q


def _sc_mesh():
    return plsc.VectorSubcoreMesh(core_axis_name="c", subcore_axis_name="s")


def _sc_post_body(sim_hbm, selT_hbm, labels_hbm, qry_hbm, tgt_hbm,
                  idsx_hbm, rankx_hbm,
                  selT_v, idsT_v, lab_v, keys_v, stage_v, rankall_v,
                  qry_v, tgt_v, cand_v, sem):
    c = lax.axis_index("c")
    s = lax.axis_index("s")
    iota = lax.iota(jnp.int32, 16)

    @pl.when(c == 0)
    def _():
        pltpu.sync_copy(selT_hbm, selT_v)

        def col_body(cc, _):
            col = s * 8 + cc
            colv = jnp.full((16,), 1, jnp.int32) * col
            cand_rows = []
            copies = []
            for k in range(_NCAND // 16):
                j = iota + k * 16
                tt = j >> 3
                jj = j & (_NSLAB - 1)
                selg = plsc.load_gather(selT_v, [col * 16 + tt])
                row = (selg & (_NG - 1)) + (jj << 12)
                cand_rows.append(row)
                pltpu.async_copy(
                    sim_hbm.at[row], cand_v.at[pl.ds(k * 16, 16)], sem).wait()
            vs = []
            for k in range(_NCAND // 16):
                vs.append(plsc.load_gather(cand_v, [iota + k * 16, colv]))

            def sel_body(t2, carry):
                mp, qp, resq = carry
                mv = jnp.where((vs[0] < mp) | ((vs[0] == mp) & (cand_rows[0] > qp)),
                               vs[0], -jnp.inf)
                mi = cand_rows[0]
                for k in range(1, _NCAND // 16):
                    ev = jnp.where((vs[k] < mp) | ((vs[k] == mp) & (cand_rows[k] > qp)),
                                   vs[k], -jnp.inf)
                    take = (ev > mv) | ((ev == mv) & (cand_rows[k] < mi))
                    mi = jnp.where(take, cand_rows[k], mi)
                    mv = jnp.where(take, ev, mv)
                m = jnp.max(mv)
                q = jnp.min(jnp.where(mv == m, mi, 2**30))
                resq = jnp.where(iota == t2, q, resq)
                return m, q, resq

            _, _, resq = lax.fori_loop(
                0, _TOPK, sel_body,
                (jnp.float32(jnp.inf), jnp.int32(-1), jnp.zeros((16,), jnp.int32)))
            stage_v[...] = resq
            pltpu.sync_copy(stage_v, idsx_hbm.at[pl.ds(col * 16, 16)])
            return 0

        lax.fori_loop(0, _NT // 16, col_body, 0)
        plsc.subcore_barrier()
        pltpu.sync_copy(idsx_hbm, idsT_v)

        lo = s * _QSLICE

        zeros16 = jnp.zeros((16,), jnp.int32)

        def zero_body(i, _):
            plsc.store_scatter(lab_v, [iota + i * 16], zeros16)
            return 0
        lax.fori_loop(0, _QSLICE // 16, zero_body, 0)

        def scatter_phase(col, carry):
            neg_phase = carry
            ids16 = plsc.load_gather(idsT_v, [col * 16 + iota])
            ids16 = ids16 & (_NQ - 1)
            inrange = (ids16 >= lo) & (ids16 < lo + _QSLICE)
            lane_ok = jnp.where(neg_phase == 1, (iota >= _POS) & (iota < _TOPK),
                                iota < _POS)
            val = jnp.full((16,), 0, jnp.int32) + jnp.where(neg_phase == 1, -1, 1)
            plsc.store_scatter(lab_v, [ids16 - lo], val, mask=inrange & lane_ok)
            return carry

        lax.fori_loop(0, _NT, scatter_phase, jnp.int32(1))
        lax.fori_loop(0, _NT, scatter_phase, jnp.int32(0))
        pltpu.sync_copy(lab_v, labels_hbm.at[pl.ds(lo, _QSLICE)])

        def key_body(k, _):
            j = iota + k * 16
            colv = j // _POS
            rankv = j - colv * _POS
            g = plsc.load_gather(idsT_v, [colv * 16 + rankv])
            plsc.store_scatter(keys_v, [j], g * _NT + colv)
            return 0
        lax.fori_loop(0, _N_PAIR // 16, key_body, 0)

        def rank_one(k):
            a = plsc.load_gather(keys_v, [iota + k * 16])

            def inner(t, acc):
                b = t // 16
                r = t - b * 16
                rot = plsc.load_gather(keys_v, [b * 16 + ((iota + r) & 15)])
                return acc + jnp.where(rot < a, 1, 0).astype(jnp.int32)

            rank = lax.fori_loop(0, _N_PAIR, inner, jnp.zeros((16,), jnp.int32))
            stage_v[...] = rank
            pltpu.sync_copy(stage_v, rankx_hbm.at[pl.ds(k * 16, 16)])

        n_vec = _N_PAIR // 16
        for m in range((n_vec + 15) // 16):
            k = s + m * 16

            @pl.when(k < n_vec)
            def _():
                rank_one(k)

        plsc.subcore_barrier()

        @pl.when(s == 0)
        def _():
            pltpu.sync_copy(rankx_hbm, rankall_v)

            def place(k, _):
                keys = plsc.load_gather(keys_v, [iota + k * 16])
                rk = plsc.load_gather(rankall_v, [iota + k * 16])
                plsc.store_scatter(qry_v, [rk], keys >> 7)
                plsc.store_scatter(tgt_v, [rk], keys & (_NT - 1))
                return 0
            lax.fori_loop(0, n_vec, place, 0)
            pltpu.sync_copy(qry_v, qry_hbm)
            pltpu.sync_copy(tgt_v, tgt_hbm)


def _sc_post(sim_matrix, selT):
    fn = functools.partial(
        pl.kernel,
        mesh=_sc_mesh(),
        out_type=[jax.ShapeDtypeStruct((_NQ,), jnp.int32),
                  jax.ShapeDtypeStruct((_N_PAIR,), jnp.int32),
                  jax.ShapeDtypeStruct((_N_PAIR,), jnp.int32),
                  jax.ShapeDtypeStruct((_NT * 16,), jnp.int32),
                  jax.ShapeDtypeStruct((_N_PAIR,), jnp.int32)],
        scratch_types=[pltpu.VMEM((_NT * 16,), jnp.int32),
                       pltpu.VMEM((_NT * 16,), jnp.int32),
                       pltpu.VMEM((_QSLICE,), jnp.int32),
                       pltpu.VMEM((_N_PAIR,), jnp.int32),
                       pltpu.VMEM((16,), jnp.int32),
                       pltpu.VMEM((_N_PAIR,), jnp.int32),
                       pltpu.VMEM((_N_PAIR,), jnp.int32),
                       pltpu.VMEM((_N_PAIR,), jnp.int32),
                       pltpu.VMEM((_NCAND, _NT), jnp.float32),
                       pltpu.SemaphoreType.DMA],
        compiler_params=pltpu.CompilerParams(needs_layout_passes=False),
    )(_sc_post_body)
    labels, qry, tgt, _ids_x, _rank_x = fn(sim_matrix, selT)
    return labels, qry, tgt


def kernel(sim_matrix):
    sel = pl.pallas_call(
        _gsel_kernel,
        grid=(_TOPK,),
        in_specs=[pl.BlockSpec((_NQ, _NT), lambda t: (0, 0))],
        out_specs=pl.BlockSpec((16, _NT), lambda t: (0, 0)),
        out_shape=jax.ShapeDtypeStruct((16, _NT), jnp.int32),
        scratch_shapes=[pltpu.VMEM((_NG, _NT), jnp.float32),
                        pltpu.VMEM((_NG, _NT), jnp.int32),
                        pltpu.VMEM((1, _NT), jnp.float32),
                        pltpu.VMEM((1, _NT), jnp.int32)],
    )(sim_matrix)

    selT = sel.T.reshape(_NT * 16)
    labels, qry, tgt = _sc_post(sim_matrix, selT)
    return labels, qry, tgt


# speedup vs baseline: 13.3872x; 1.0017x over previous
import functools

import jax
import jax.numpy as jnp
from jax import lax
from jax.experimental import pallas as pl
from jax.experimental.pallas import tpu as pltpu
from jax.experimental.pallas import tpu_sc as plsc

_NQ = 32768
_NT = 128
_TOPK = 10
_POS = 5
_NG = 4096
_NSLAB = _NQ // _NG
_MCHUNK = 256

_N_PAIR = _POS * _NT
_QSLICE = _NQ // 16
_NCAND = _TOPK * _NSLAB


def _gsel_kernel(sim_ref, sel_ref, m_ref, m2_ref, mprev_ref, qprev_ref):
    t = pl.program_id(0)

    @pl.when(t == 0)
    def _():
        mprev_ref[...] = jnp.full((1, _NT), jnp.inf, jnp.float32)
        qprev_ref[...] = jnp.full((1, _NT), -1, jnp.int32)

        def mbody(i, _):
            base = i * _MCHUNK
            giota = jax.lax.broadcasted_iota(jnp.int32, (_MCHUNK, _NT), 0) + base
            cur = sim_ref[pl.ds(base, _MCHUNK), :]
            curi = giota
            for j in range(1, _NSLAB):
                blk = sim_ref[pl.ds(base + j * _NG, _MCHUNK), :]
                take = blk > cur
                curi = jnp.where(take, giota + j * _NG, curi)
                cur = jnp.where(take, blk, cur)
            m_ref[pl.ds(base, _MCHUNK), :] = cur
            m2_ref[pl.ds(base, _MCHUNK), :] = curi
            return 0

        lax.fori_loop(0, _NG // _MCHUNK, mbody, 0)

    mp = mprev_ref[...]
    qp = qprev_ref[...]

    def body(i, carry):
        m, q = carry
        blk = m_ref[pl.ds(i * _MCHUNK, _MCHUNK), :]
        bli = m2_ref[pl.ds(i * _MCHUNK, _MCHUNK), :]
        elig = (blk < mp) | ((blk == mp) & (bli > qp))
        mb = jnp.where(elig, blk, -jnp.inf)
        bm = jnp.max(mb, axis=0, keepdims=True)
        bq = jnp.min(jnp.where(mb == bm, bli, 2**30), axis=0, keepdims=True)
        take = (bm > m) | ((bm == m) & (bq < q))
        return jnp.where(take, bm, m), jnp.where(take, bq, q)

    m0 = jnp.full((1, _NT), -jnp.inf, jnp.float32)
    q0 = jnp.full((1, _NT), 2**30, jnp.int32)
    m, q = lax.fori_loop(0, _NG // _MCHUNK, body, (m0, q0))
    sel_ref[pl.ds(t, 1), :] = q
    mprev_ref[...] = m
    qprev_ref[...] = q


def _sc_mesh():
    return plsc.VectorSubcoreMesh(core_axis_name="c", subcore_axis_name="s")


def _sc_post_body(sim_hbm, selT_hbm, labels_hbm, qry_hbm, tgt_hbm,
                  idsx_hbm, rankx_hbm,
                  selT_v, idsT_v, lab_v, keys_v, stage_v, rankall_v,
                  qry_v, tgt_v, cand_v, sem):
    c = lax.axis_index("c")
    s = lax.axis_index("s")
    iota = lax.iota(jnp.int32, 16)

    @pl.when(c == 0)
    def _():
        pltpu.sync_copy(selT_hbm, selT_v)

        def col_body(cc, _):
            col = s * 8 + cc
            colv = jnp.full((16,), 1, jnp.int32) * col
            cand_rows = []
            for k in range(_NCAND // 16):
                j = iota + k * 16
                tt = j >> 3
                jj = j & (_NSLAB - 1)
                selg = plsc.load_gather(selT_v, [col * 16 + tt])
                row = (selg & (_NG - 1)) + (jj << 12)
                cand_rows.append(row)
                pltpu.async_copy(
                    sim_hbm.at[row], cand_v.at[pl.ds(k * 16, 16)], sem).wait()
            vs = []
            for k in range(_NCAND // 16):
                vs.append(plsc.load_gather(cand_v, [iota + k * 16, colv]))

            def sel_body(t2, carry):
                mp, qp, resq = carry
                mv = jnp.where((vs[0] < mp) | ((vs[0] == mp) & (cand_rows[0] > qp)),
                               vs[0], -jnp.inf)
                mi = cand_rows[0]
                for k in range(1, _NCAND // 16):
                    ev = jnp.where((vs[k] < mp) | ((vs[k] == mp) & (cand_rows[k] > qp)),
                                   vs[k], -jnp.inf)
                    take = (ev > mv) | ((ev == mv) & (cand_rows[k] < mi))
                    mi = jnp.where(take, cand_rows[k], mi)
                    mv = jnp.where(take, ev, mv)
                m = jnp.max(mv)
                q = jnp.min(jnp.where(mv == m, mi, 2**30))
                resq = jnp.where(iota == t2, q, resq)
                return m, q, resq

            _, _, resq = lax.fori_loop(
                0, _TOPK, sel_body,
                (jnp.float32(jnp.inf), jnp.int32(-1), jnp.zeros((16,), jnp.int32)))
            stage_v[...] = resq
            pltpu.sync_copy(stage_v, idsx_hbm.at[pl.ds(col * 16, 16)])
            return 0

        lax.fori_loop(0, _NT // 16, col_body, 0)
        plsc.subcore_barrier()
        pltpu.sync_copy(idsx_hbm, idsT_v)

        lo = s * _QSLICE

        zeros16 = jnp.zeros((16,), jnp.int32)

        def zero_body(i, _):
            plsc.store_scatter(lab_v, [iota + i * 16], zeros16)
            return 0
        lax.fori_loop(0, _QSLICE // 16, zero_body, 0)

        def scatter_phase(col, carry):
            neg_phase = carry
            ids16 = plsc.load_gather(idsT_v, [col * 16 + iota])
            ids16 = ids16 & (_NQ - 1)
            inrange = (ids16 >= lo) & (ids16 < lo + _QSLICE)
            lane_ok = jnp.where(neg_phase == 1, (iota >= _POS) & (iota < _TOPK),
                                iota < _POS)
            val = jnp.full((16,), 0, jnp.int32) + jnp.where(neg_phase == 1, -1, 1)
            plsc.store_scatter(lab_v, [ids16 - lo], val, mask=inrange & lane_ok)
            return carry

        lax.fori_loop(0, _NT, scatter_phase, jnp.int32(1))
        lax.fori_loop(0, _NT, scatter_phase, jnp.int32(0))
        pltpu.sync_copy(lab_v, labels_hbm.at[pl.ds(lo, _QSLICE)])

        def key_body(k, _):
            j = iota + k * 16
            colv = j // _POS
            rankv = j - colv * _POS
            g = plsc.load_gather(idsT_v, [colv * 16 + rankv])
            plsc.store_scatter(keys_v, [j], g * _NT + colv)
            return 0
        lax.fori_loop(0, _N_PAIR // 16, key_body, 0)

        def rank_one(k):
            a = plsc.load_gather(keys_v, [iota + k * 16])

            def inner(t, acc):
                b = t // 16
                r = t - b * 16
                rot = plsc.load_gather(keys_v, [b * 16 + ((iota + r) & 15)])
                return acc + jnp.where(rot < a, 1, 0).astype(jnp.int32)

            rank = lax.fori_loop(0, _N_PAIR, inner, jnp.zeros((16,), jnp.int32))
            stage_v[...] = rank
            pltpu.sync_copy(stage_v, rankx_hbm.at[pl.ds(k * 16, 16)])

        n_vec = _N_PAIR // 16
        for m in range((n_vec + 15) // 16):
            k = s + m * 16

            @pl.when(k < n_vec)
            def _():
                rank_one(k)

        plsc.subcore_barrier()

        @pl.when(s == 0)
        def _():
            pltpu.sync_copy(rankx_hbm, rankall_v)

            def place(k, _):
                keys = plsc.load_gather(keys_v, [iota + k * 16])
                rk = plsc.load_gather(rankall_v, [iota + k * 16])
                plsc.store_scatter(qry_v, [rk], keys >> 7)
                plsc.store_scatter(tgt_v, [rk], keys & (_NT - 1))
                return 0
            lax.fori_loop(0, n_vec, place, 0)
            pltpu.sync_copy(qry_v, qry_hbm)
            pltpu.sync_copy(tgt_v, tgt_hbm)


def _sc_post(sim_matrix, selT):
    fn = functools.partial(
        pl.kernel,
        mesh=_sc_mesh(),
        out_type=[jax.ShapeDtypeStruct((_NQ,), jnp.int32),
                  jax.ShapeDtypeStruct((_N_PAIR,), jnp.int32),
                  jax.ShapeDtypeStruct((_N_PAIR,), jnp.int32),
                  jax.ShapeDtypeStruct((_NT * 16,), jnp.int32),
                  jax.ShapeDtypeStruct((_N_PAIR,), jnp.int32)],
        scratch_types=[pltpu.VMEM((_NT * 16,), jnp.int32),
                       pltpu.VMEM((_NT * 16,), jnp.int32),
                       pltpu.VMEM((_QSLICE,), jnp.int32),
                       pltpu.VMEM((_N_PAIR,), jnp.int32),
                       pltpu.VMEM((16,), jnp.int32),
                       pltpu.VMEM((_N_PAIR,), jnp.int32),
                       pltpu.VMEM((_N_PAIR,), jnp.int32),
                       pltpu.VMEM((_N_PAIR,), jnp.int32),
                       pltpu.VMEM((_NCAND, _NT), jnp.float32),
                       pltpu.SemaphoreType.DMA],
        compiler_params=pltpu.CompilerParams(needs_layout_passes=False),
    )(_sc_post_body)
    labels, qry, tgt, _ids_x, _rank_x = fn(sim_matrix, selT)
    return labels, qry, tgt


def kernel(sim_matrix):
    sel = pl.pallas_call(
        _gsel_kernel,
        grid=(_TOPK,),
        in_specs=[pl.BlockSpec((_NQ, _NT), lambda t: (0, 0))],
        out_specs=pl.BlockSpec((16, _NT), lambda t: (0, 0)),
        out_shape=jax.ShapeDtypeStruct((16, _NT), jnp.int32),
        scratch_shapes=[pltpu.VMEM((_NG, _NT), jnp.float32),
                        pltpu.VMEM((_NG, _NT), jnp.int32),
                        pltpu.VMEM((1, _NT), jnp.float32),
                        pltpu.VMEM((1, _NT), jnp.int32)],
    )(sim_matrix)

    selT = sel.T.reshape(_NT * 16)
    labels, qry, tgt = _sc_post(sim_matrix, selT)
    return labels, qry, tgt
